# Initial kernel scaffold; baseline (speedup 1.0000x reference)
import jax
import jax.numpy as jnp
from jax.experimental import pallas as pl


def kernel(input_ids, table):
    raise NotImplementedError("write your pallas kernel here")


# speedup vs baseline: 3.3545x; 3.3545x over previous
import functools

import jax
import jax.numpy as jnp
from jax import lax
from jax.experimental import pallas as pl
from jax.experimental.pallas import tpu as pltpu
from jax.experimental.pallas import tpu_sc as plsc

_IDXW = 128
_GPC = 4
_CHUNK = _IDXW * _GPC
_DP = 56


def _make_gather(n_flat: int):
    info = plsc.get_sparse_core_info()
    nw = info.num_cores * info.num_subcores
    assert n_flat % (nw * _CHUNK) == 0
    per_w = n_flat // nw
    n_rows_w = per_w // _IDXW
    n_outer = per_w // _CHUNK

    mesh = plsc.VectorSubcoreMesh(core_axis_name="c", subcore_axis_name="s")

    @functools.partial(
        pl.kernel,
        out_type=jax.ShapeDtypeStruct((n_flat, _DP), jnp.float32),
        mesh=mesh,
        compiler_params=pltpu.CompilerParams(use_tc_tiling_on_sc=False),
        scratch_types=[
            pltpu.VMEM((n_rows_w, _IDXW), jnp.int32),
            pltpu.VMEM((_CHUNK, _DP), jnp.float32),
            pltpu.SemaphoreType.DMA,
        ],
    )
    def gather_kernel(idx_hbm, table_hbm, out_hbm, idx_v, rows_v, sem):
        wid = lax.axis_index("s") * info.num_cores + lax.axis_index("c")
        pltpu.sync_copy(idx_hbm.at[pl.ds(wid * n_rows_w, n_rows_w)], idx_v)

        def body(c, _):
            copies = []
            for g in range(_GPC):
                cp = pltpu.make_async_copy(
                    table_hbm.at[idx_v.at[c * _GPC + g]],
                    rows_v.at[pl.ds(g * _IDXW, _IDXW)],
                    sem,
                )
                cp.start()
                copies.append(cp)
            for cp in copies:
                cp.wait()
            base = wid * per_w + c * _CHUNK
            pltpu.sync_copy(rows_v, out_hbm.at[pl.ds(base, _CHUNK)])
            return ()

        lax.fori_loop(0, n_outer, body, (), unroll=False)

    return gather_kernel


def kernel(input_ids, table):
    b, l = input_ids.shape
    vocab, dim = table.shape
    n_flat = b * l
    tpad = jnp.pad(table, ((0, 0), (0, _DP - dim)))
    idx = input_ids.reshape(n_flat // _IDXW, _IDXW)
    outp = _make_gather(n_flat)(idx, tpad)
    return outp[:, :dim].reshape(b, l, dim)


# speedup vs baseline: 3.4130x; 1.0174x over previous
import functools

import jax
import jax.numpy as jnp
from jax import lax
from jax.experimental import pallas as pl
from jax.experimental.pallas import tpu as pltpu
from jax.experimental.pallas import tpu_sc as plsc

_IDXW = 128
_GPC = 8
_CHUNK = _IDXW * _GPC
_DP = 56


def _make_gather(n_flat: int):
    info = plsc.get_sparse_core_info()
    nw = info.num_cores * info.num_subcores
    assert n_flat % (nw * _CHUNK) == 0
    per_w = n_flat // nw
    n_rows_w = per_w // _IDXW
    n_outer = per_w // _CHUNK

    mesh = plsc.VectorSubcoreMesh(core_axis_name="c", subcore_axis_name="s")

    @functools.partial(
        pl.kernel,
        out_type=jax.ShapeDtypeStruct((n_flat, _DP), jnp.float32),
        mesh=mesh,
        compiler_params=pltpu.CompilerParams(use_tc_tiling_on_sc=False),
        scratch_types=[
            pltpu.VMEM((n_rows_w, _IDXW), jnp.int32),
            pltpu.VMEM((_CHUNK, _DP), jnp.float32),
            pltpu.SemaphoreType.DMA,
        ],
    )
    def gather_kernel(idx_hbm, table_hbm, out_hbm, idx_v, rows_v, sem):
        wid = lax.axis_index("s") * info.num_cores + lax.axis_index("c")
        pltpu.sync_copy(idx_hbm.at[pl.ds(wid * n_rows_w, n_rows_w)], idx_v)

        def body(c, _):
            copies = []
            for g in range(_GPC):
                cp = pltpu.make_async_copy(
                    table_hbm.at[idx_v.at[c * _GPC + g]],
                    rows_v.at[pl.ds(g * _IDXW, _IDXW)],
                    sem,
                )
                cp.start()
                copies.append(cp)
            for cp in copies:
                cp.wait()
            base = wid * per_w + c * _CHUNK
            pltpu.sync_copy(rows_v, out_hbm.at[pl.ds(base, _CHUNK)])
            return ()

        lax.fori_loop(0, n_outer, body, (), unroll=False)

    return gather_kernel


def kernel(input_ids, table):
    b, l = input_ids.shape
    vocab, dim = table.shape
    n_flat = b * l
    tpad = jnp.pad(table, ((0, 0), (0, _DP - dim)))
    idx = input_ids.reshape(n_flat // _IDXW, _IDXW)
    outp = _make_gather(n_flat)(idx, tpad)
    return outp[:, :dim].reshape(b, l, dim)


# speedup vs baseline: 3.4310x; 1.0053x over previous
import functools

import jax
import jax.numpy as jnp
from jax import lax
from jax.experimental import pallas as pl
from jax.experimental.pallas import tpu as pltpu
from jax.experimental.pallas import tpu_sc as plsc

_IDXW = 128
_GPC = 8
_CHUNK = _IDXW * _GPC
_DP = 56


def _make_gather(n_flat: int):
    info = plsc.get_sparse_core_info()
    nw = info.num_cores * info.num_subcores
    assert n_flat % (nw * _CHUNK) == 0
    per_w = n_flat // nw
    n_rows_w = per_w // _IDXW
    n_outer = per_w // _CHUNK

    mesh = plsc.VectorSubcoreMesh(core_axis_name="c", subcore_axis_name="s")

    @functools.partial(
        pl.kernel,
        out_type=jax.ShapeDtypeStruct((n_flat, _DP), jnp.float32),
        mesh=mesh,
        compiler_params=pltpu.CompilerParams(use_tc_tiling_on_sc=False),
        scratch_types=[
            pltpu.VMEM((n_rows_w, _IDXW), jnp.int32),
            pltpu.VMEM((_CHUNK, _DP), jnp.float32),
        ] + [pltpu.SemaphoreType.DMA] * _GPC,
    )
    def gather_kernel(idx_hbm, table_hbm, out_hbm, idx_v, rows_v, *sems):
        wid = lax.axis_index("s") * info.num_cores + lax.axis_index("c")
        pltpu.sync_copy(idx_hbm.at[pl.ds(wid * n_rows_w, n_rows_w)], idx_v)

        def body(c, _):
            copies = []
            for g in range(_GPC):
                cp = pltpu.make_async_copy(
                    table_hbm.at[idx_v.at[c * _GPC + g]],
                    rows_v.at[pl.ds(g * _IDXW, _IDXW)],
                    sems[g],
                )
                cp.start()
                copies.append(cp)
            for cp in copies:
                cp.wait()
            base = wid * per_w + c * _CHUNK
            pltpu.sync_copy(rows_v, out_hbm.at[pl.ds(base, _CHUNK)])
            return ()

        lax.fori_loop(0, n_outer, body, (), unroll=False)

    return gather_kernel


def kernel(input_ids, table):
    b, l = input_ids.shape
    vocab, dim = table.shape
    n_flat = b * l
    tpad = jnp.pad(table, ((0, 0), (0, _DP - dim)))
    idx = input_ids.reshape(n_flat // _IDXW, _IDXW)
    outp = _make_gather(n_flat)(idx, tpad)
    return outp[:, :dim].reshape(b, l, dim)
